# baseline (speedup 1.0000x reference)
import functools

import jax
import jax.numpy as jnp
from jax import lax
from jax.experimental import pallas as pl
from jax.experimental.pallas import tpu as pltpu
from jax.experimental.pallas import tpu_sc as plsc

N = 1000000
D = 64
B = 16384
NW = 32
SPW = 245
WSPAN = SPW * 128
FULL_END = (N // 128) * 128
TAIL = N - FULL_END
CHUNK = 384
NCHUNKS = 82
ALIGN_MAX = FULL_END - CHUNK
DUMP = B
WIN = 2048
GPB = 8


def kernel(node_ids, table):
    mesh = plsc.VectorSubcoreMesh(core_axis_name="c", subcore_axis_name="s")

    @functools.partial(
        pl.kernel,
        mesh=mesh,
        out_type=jax.ShapeDtypeStruct((B + 16, 128), jnp.float32),
        scratch_types=[
            pltpu.VMEM((B,), jnp.int32),
            pltpu.VMEM((B + 16,), jnp.int32),
            pltpu.VMEM((B + 16,), jnp.int32),
            pltpu.VMEM((WIN + 16,), jnp.int32),
            pltpu.VMEM((WIN + 16,), jnp.int32),
            pltpu.VMEM((D, CHUNK), jnp.float32),
            pltpu.VMEM((D, CHUNK), jnp.float32),
            pltpu.VMEM((D, TAIL), jnp.float32),
            pltpu.VMEM((16 * GPB, 128), jnp.float32),
            pltpu.VMEM((1, 16 * GPB), jnp.int32),
            pltpu.SemaphoreType.DMA,
            pltpu.SemaphoreType.DMA,
            pltpu.SemaphoreType.DMA,
        ],
        compiler_params=pltpu.CompilerParams(needs_layout_passes=False),
    )
    def k1(idx_hbm, tableT_hbm, tailT_hbm, temp_hbm, ids_v, mid_v, mrow_v,
           cid_v, crow_v, sb0, sb1, tail_v, ostage, ridx_v, sem0, sem1,
           rsem):
        wid = lax.axis_index("s") * 2 + lax.axis_index("c")
        lo = wid * WSPAN
        hi = jnp.minimum(lo + WSPAN, N)
        hi_full = jnp.minimum(lo + WSPAN, FULL_END)
        iota = lax.iota(jnp.int32, 16)

        pltpu.sync_copy(idx_hbm, ids_v)
        pltpu.sync_copy(tailT_hbm, tail_v)

        def scan_body(g, cnt):
            ids = ids_v[pl.ds(g * 16, 16)]
            m = (ids >= lo) & (ids < hi)
            plsc.store_compressed(mid_v.at[pl.ds(cnt, 16)], ids, mask=m)
            plsc.store_compressed(mrow_v.at[pl.ds(cnt, 16)], iota + g * 16,
                                  mask=m)
            return cnt + jnp.max(plsc.all_reduce_population_count(m))

        mcnt = lax.fori_loop(0, B // 16, scan_body, jnp.int32(0))

        def flush():
            pltpu.async_copy(ostage, temp_hbm.at[ridx_v.at[0]], rsem).wait()

        def extract(src_ref, dma_start, cs, ce, gt0):
            def win_body(w, gt_w):
                base = w * WIN
                nvec = (jnp.minimum(mcnt - base, WIN) + 15) >> 4

                def filt(v, ccnt):
                    off = base + v * 16
                    ids = mid_v[pl.ds(off, 16)]
                    rows = mrow_v[pl.ds(off, 16)]
                    m = ((iota + off) < mcnt) & (ids >= cs) & (ids < ce)
                    plsc.store_compressed(cid_v.at[pl.ds(ccnt, 16)], ids,
                                          mask=m)
                    plsc.store_compressed(crow_v.at[pl.ds(ccnt, 16)], rows,
                                          mask=m)
                    return ccnt + jnp.max(plsc.all_reduce_population_count(m))

                ccnt = lax.fori_loop(0, nvec, filt, jnp.int32(0))

                def grp(g, gt):
                    gl = gt & (GPB - 1)

                    @pl.when(gl == 0)
                    def _clear_ridx():
                        for s in range(GPB):
                            ridx_v[0, pl.ds(s * 16, 16)] = jnp.full(
                                (16,), DUMP, jnp.int32)

                    ids = cid_v[pl.ds(g * 16, 16)]
                    rows = crow_v[pl.ds(g * 16, 16)]
                    lm = iota < (ccnt - g * 16)
                    pos = jnp.where(lm, ids - dma_start, 0)
                    rvec = gl * 16 + iota
                    ridx_v[0, pl.ds(gl * 16, 16)] = jnp.where(lm, rows, DUMP)
                    for d in range(D):
                        dv = jnp.full((16,), d, jnp.int32)
                        vals = plsc.load_gather(src_ref, [dv, pos])
                        plsc.store_scatter(ostage, [rvec, dv], vals)

                    @pl.when(gl == GPB - 1)
                    def _flush_full():
                        flush()

                    return gt + 1

                return lax.fori_loop(0, (ccnt + 15) >> 4, grp, gt_w)

            return lax.fori_loop(0, (mcnt + (WIN - 1)) >> 11, win_body, gt0)

        def issue(c, sbuf, sem):
            dma_cs = jnp.minimum(lo + c * CHUNK, ALIGN_MAX)
            pltpu.async_copy(tableT_hbm.at[:, pl.ds(dma_cs, CHUNK)], sbuf,
                             sem)

        issue(jnp.int32(0), sb0, sem0)
        issue(jnp.int32(1), sb1, sem1)

        def pair(i, gt):
            for b, (sbuf, sem) in enumerate(((sb0, sem0), (sb1, sem1))):
                c = i * 2 + b
                dma_cs = jnp.minimum(lo + c * CHUNK, ALIGN_MAX)
                pltpu.make_async_copy(
                    tableT_hbm.at[:, pl.ds(dma_cs, CHUNK)], sbuf, sem).wait()
                cs = lo + c * CHUNK
                ce = jnp.minimum(cs + CHUNK, hi_full)
                gt = extract(sbuf, dma_cs, cs, ce, gt)

                @pl.when(c + 2 < NCHUNKS)
                def _issue_next(sbuf=sbuf, sem=sem, c=c):
                    issue(c + 2, sbuf, sem)

            return gt

        gt = lax.fori_loop(0, NCHUNKS // 2, pair, jnp.int32(0))

        gt = extract(tail_v, jnp.int32(FULL_END), jnp.int32(FULL_END),
                     jnp.int32(N), gt)

        @pl.when((gt & (GPB - 1)) != 0)
        def _flush_tail():
            flush()

    tailT = lax.slice(table, (FULL_END, 0), (N, D)).T
    temp = k1(node_ids.astype(jnp.int32), table.T, tailT)
    return temp[:B, :D]


# speedup vs baseline: 5.6787x; 5.6787x over previous
import functools

import jax
import jax.numpy as jnp
from jax import lax
from jax.experimental import pallas as pl
from jax.experimental.pallas import tpu as pltpu
from jax.experimental.pallas import tpu_sc as plsc

N = 1000000
D = 64
B = 16384
NW = 32
SPW = 245
WSPAN = SPW * 128
FULL_END = (N // 128) * 128
TAIL = N - FULL_END
CHUNK = 256
NCHUNKS = 124
ALIGN_MAX = FULL_END - CHUNK
WIN = 2048
RCAP = B + 16 * 128
TROWS = NW * RCAP
DUMP = B
OCAP = 144


def kernel(node_ids, table):
    mesh = plsc.VectorSubcoreMesh(core_axis_name="c", subcore_axis_name="s")

    @functools.partial(
        pl.kernel,
        mesh=mesh,
        out_type=[
            jax.ShapeDtypeStruct((TROWS * 128,), jnp.float32),
            jax.ShapeDtypeStruct((TROWS,), jnp.int32),
            jax.ShapeDtypeStruct((NW * 16,), jnp.int32),
        ],
        scratch_types=[
            pltpu.VMEM((B + 16,), jnp.int32),
            pltpu.VMEM((B + 16,), jnp.int32),
            pltpu.VMEM((WIN + 16,), jnp.int32),
            pltpu.VMEM((WIN + 16,), jnp.int32),
            pltpu.VMEM((D, CHUNK), jnp.float32),
            pltpu.VMEM((D, CHUNK), jnp.float32),
            pltpu.VMEM((D, TAIL), jnp.float32),
            pltpu.VMEM((OCAP * 128,), jnp.float32),
            pltpu.VMEM((OCAP,), jnp.int32),
            pltpu.VMEM((16,), jnp.int32),
            pltpu.SemaphoreType.DMA,
            pltpu.SemaphoreType.DMA,
        ],
        compiler_params=pltpu.CompilerParams(needs_layout_passes=False),
    )
    def k1(idx_hbm, tableT_hbm, tailT_hbm, temp_hbm, rowmap_hbm, counts_hbm,
           mid_v, mrow_v, cid_v, crow_v, sb0, sb1, tail_v, ostage,
           oridx, stage16, sem0, sem1):
        half = lax.axis_index("c")
        sid = lax.axis_index("s")
        wid = sid * 2 + half
        lo = wid * WSPAN
        hi = jnp.minimum(lo + WSPAN, N)
        hi_full = jnp.minimum(lo + WSPAN, FULL_END)
        iota = lax.iota(jnp.int32, 16)

        pltpu.sync_copy(tailT_hbm, tail_v)

        def scan_blk(bk, cnt):
            pltpu.sync_copy(
                idx_hbm.at[pl.ds(pl.multiple_of(bk * 2048, 2048), 2048)],
                cid_v.at[pl.ds(0, 2048)])

            def scan_body(g, cnt):
                ids = cid_v[pl.ds(g * 16, 16)]
                m = (ids >= lo) & (ids < hi)
                plsc.store_compressed(mid_v.at[pl.ds(cnt, 16)], ids, mask=m)
                plsc.store_compressed(
                    mrow_v.at[pl.ds(cnt, 16)],
                    iota + (bk * 2048 + g * 16), mask=m)
                return cnt + jnp.max(plsc.all_reduce_population_count(m))

            return lax.fori_loop(0, 128, scan_body, cnt)

        mcnt = lax.fori_loop(0, B // 2048, scan_blk, jnp.int32(0))

        padded = (mcnt + 127) & ~jnp.int32(127)
        stage16[...] = jnp.full((16,), 0, jnp.int32) + padded
        pltpu.sync_copy(
            stage16,
            counts_hbm.at[pl.ds(pl.multiple_of(wid * 16, 16), 16)])
        region = wid * RCAP

        def flush(fbase):
            rowstart = pl.multiple_of(
                jnp.minimum(region + fbase * 128, TROWS - 128), 128)
            pltpu.sync_copy(ostage.at[pl.ds(0, 128 * 128)],
                            temp_hbm.at[pl.ds(rowstart * 128, 128 * 128)])
            pltpu.sync_copy(oridx.at[pl.ds(0, 128)],
                            rowmap_hbm.at[pl.ds(rowstart, 128)])

        def extract(src_ref, dma_start, cs, ce, st0):
            def win_body(w, st_w):
                bb = w * WIN
                nvec = (jnp.minimum(mcnt - bb, WIN) + 15) >> 4

                def filt(v, ccnt):
                    off = bb + v * 16
                    ids = mid_v[pl.ds(off, 16)]
                    rows = mrow_v[pl.ds(off, 16)]
                    m = ((iota + off) < mcnt) & (ids >= cs) & (ids < ce)
                    plsc.store_compressed(cid_v.at[pl.ds(ccnt, 16)], ids,
                                          mask=m)
                    plsc.store_compressed(crow_v.at[pl.ds(ccnt, 16)], rows,
                                          mask=m)
                    return ccnt + jnp.max(plsc.all_reduce_population_count(m))

                ccnt = lax.fori_loop(0, nvec, filt, jnp.int32(0))

                def grp(g, st):
                    acnt, fbase = st
                    ids = cid_v[pl.ds(g * 16, 16)]
                    rows = crow_v[pl.ds(g * 16, 16)]
                    lm = iota < (ccnt - g * 16)
                    pos = jnp.where(lm, ids - dma_start, 0)
                    prow = jnp.minimum(
                        acnt + plsc.cumsum(lm.astype(jnp.int32)) - 1,
                        OCAP - 1)
                    plsc.store_scatter(oridx, [prow], rows, mask=lm)
                    pbase = prow * 128
                    for d in range(D):
                        dv = jnp.full((16,), d, jnp.int32)
                        vals = plsc.load_gather(src_ref, [dv, pos])
                        plsc.store_scatter(ostage, [pbase + d], vals,
                                           mask=lm)
                    nacnt = acnt + jnp.max(
                        plsc.all_reduce_population_count(lm))
                    do_flush = nacnt >= 128

                    @pl.when(do_flush)
                    def _flush_and_carry():
                        flush(fbase)
                        rem = nacnt - 128

                        def move_row(k, _):
                            src = (128 + k) * 128
                            dst = k * 128
                            for t in range(8):
                                ostage[pl.ds(dst + t * 16, 16)] = (
                                    ostage[pl.ds(src + t * 16, 16)])
                            return _

                        lax.fori_loop(0, rem, move_row, 0)
                        mv = oridx[pl.ds(128, 16)]
                        oridx[pl.ds(0, 16)] = mv

                    acnt = jnp.where(do_flush, nacnt - 128, nacnt)
                    fbase = jnp.where(do_flush, fbase + 1, fbase)
                    return (acnt, fbase)

                return lax.fori_loop(0, (ccnt + 15) >> 4, grp, st_w)

            return lax.fori_loop(0, (mcnt + (WIN - 1)) >> 11, win_body, st0)

        def issue(c, sbuf, sem):
            dma_cs = jnp.minimum(lo + c * CHUNK, ALIGN_MAX)
            pltpu.async_copy(tableT_hbm.at[:, pl.ds(dma_cs, CHUNK)], sbuf,
                             sem)

        issue(jnp.int32(0), sb0, sem0)
        issue(jnp.int32(1), sb1, sem1)

        def pair(i, st):
            for bb, (sbuf, sem) in enumerate(((sb0, sem0), (sb1, sem1))):
                c = i * 2 + bb
                dma_cs = jnp.minimum(lo + c * CHUNK, ALIGN_MAX)
                pltpu.make_async_copy(
                    tableT_hbm.at[:, pl.ds(dma_cs, CHUNK)], sbuf, sem).wait()
                cs = lo + c * CHUNK
                ce = jnp.minimum(cs + CHUNK, hi_full)
                st = extract(sbuf, dma_cs, cs, ce, st)

                @pl.when(c + 2 < NCHUNKS)
                def _issue_next(sbuf=sbuf, sem=sem, c=c):
                    issue(c + 2, sbuf, sem)

            return st

        st = lax.fori_loop(0, NCHUNKS // 2, pair,
                           (jnp.int32(0), jnp.int32(0)))

        acnt, fbase = extract(tail_v, jnp.int32(FULL_END),
                              jnp.int32(FULL_END), jnp.int32(N), st)

        @pl.when(acnt > 0)
        def _final_flush():
            for s in range(8):
                l = iota + s * 16
                cur = oridx[pl.ds(s * 16, 16)]
                oridx[pl.ds(s * 16, 16)] = jnp.where(l < acnt, cur, DUMP)
            flush(fbase)

    @functools.partial(
        pl.kernel,
        mesh=mesh,
        out_type=jax.ShapeDtypeStruct((B, 128), jnp.float32),
        scratch_types=[
            pltpu.VMEM((2048,), jnp.int32),
            pltpu.VMEM((512 + 16,), jnp.int32),
            pltpu.VMEM((128, 128), jnp.float32),
            pltpu.VMEM((NW * 16,), jnp.int32),
            pltpu.SemaphoreType.DMA,
        ],
        compiler_params=pltpu.CompilerParams(
            use_tc_tiling_on_sc=False, needs_layout_passes=False),
    )
    def k2(temp_hbm, rowmap_hbm, counts_hbm, out_hbm, rbuf, inv, gbuf,
           cnt_v, sem):
        wid = lax.axis_index("s") * 2 + lax.axis_index("c")
        iota = lax.iota(jnp.int32, 16)
        rlo = wid * 512
        pltpu.sync_copy(counts_hbm, cnt_v)
        for i in range(33):
            inv[pl.ds(i * 16, 16)] = jnp.zeros((16,), jnp.int32)

        for c in range(NW):
            nent = jnp.clip(jnp.max(cnt_v[pl.ds(c * 16, 16)]), 0, RCAP)

            def blk(bb, _, c=c, nent=nent):
                pltpu.sync_copy(
                    rowmap_hbm.at[pl.ds(
                        pl.multiple_of(c * RCAP + bb * 2048, 2048), 2048)],
                    rbuf)

                def vec(v, __):
                    l = bb * 2048 + v * 16
                    rm = rbuf[pl.ds(v * 16, 16)]
                    m = ((l + iota) < nent) & (rm >= rlo) & (rm < rlo + 512)
                    eidx = c * RCAP + l + iota
                    plsc.store_scatter(
                        inv, [jnp.where(m, rm - rlo, 512)], eidx, mask=m)
                    return __

                lax.fori_loop(0, 128, vec, 0)
                return _

            lax.fori_loop(0, (nent + 2047) >> 11, blk, 0)

        for sub in range(4):
            pltpu.async_copy(
                temp_hbm.at[inv.at[pl.ds(sub * 128, 128)]], gbuf, sem).wait()
            pltpu.sync_copy(gbuf, out_hbm.at[pl.ds(rlo + sub * 128, 128)])

    tailT = lax.slice(table, (FULL_END, 0), (N, D)).T
    temp, rowmap, counts = k1(node_ids.astype(jnp.int32), table.T, tailT)
    wide = k2(temp.reshape(TROWS, 128), rowmap, counts)
    return wide[:, :D]


# speedup vs baseline: 7.2331x; 1.2737x over previous
import functools

import jax
import jax.numpy as jnp
from jax import lax
from jax.experimental import pallas as pl
from jax.experimental.pallas import tpu as pltpu
from jax.experimental.pallas import tpu_sc as plsc

N = 1000000
D = 64
B = 16384
NW = 32
SPW = 245
WSPAN = SPW * 128
FULL_END = (N // 128) * 128
TAIL = N - FULL_END
CHUNK = 256
NCHUNKS = 124
ALIGN_MAX = FULL_END - CHUNK
WIN = 2048
RCAP = B + 16 * 128
TROWS = NW * RCAP
DUMP = B
OCAP = 144


def kernel(node_ids, table):
    mesh = plsc.VectorSubcoreMesh(core_axis_name="c", subcore_axis_name="s")

    @functools.partial(
        pl.kernel,
        mesh=mesh,
        out_type=[
            jax.ShapeDtypeStruct((TROWS * 128,), jnp.float32),
            jax.ShapeDtypeStruct((TROWS,), jnp.int32),
            jax.ShapeDtypeStruct((NW * 16,), jnp.int32),
        ],
        scratch_types=[
            pltpu.VMEM((B + 16,), jnp.int32),
            pltpu.VMEM((B + 16,), jnp.int32),
            pltpu.VMEM((WIN + 16,), jnp.int32),
            pltpu.VMEM((WIN + 16,), jnp.int32),
            pltpu.VMEM((D, CHUNK), jnp.float32),
            pltpu.VMEM((D, CHUNK), jnp.float32),
            pltpu.VMEM((D, CHUNK), jnp.float32),
            pltpu.VMEM((D, CHUNK), jnp.float32),
            pltpu.VMEM((D, TAIL), jnp.float32),
            pltpu.VMEM((OCAP * 128,), jnp.float32),
            pltpu.VMEM((OCAP,), jnp.int32),
            pltpu.VMEM((16,), jnp.int32),
            pltpu.SemaphoreType.DMA,
            pltpu.SemaphoreType.DMA,
            pltpu.SemaphoreType.DMA,
            pltpu.SemaphoreType.DMA,
        ],
        compiler_params=pltpu.CompilerParams(needs_layout_passes=False),
    )
    def k1(idx_hbm, tableT_hbm, tailT_hbm, temp_hbm, rowmap_hbm, counts_hbm,
           mid_v, mrow_v, cid_v, crow_v, sb0, sb1, sb2, sb3, tail_v,
           ostage, oridx, stage16, sem0, sem1, sem2, sem3):
        half = lax.axis_index("c")
        sid = lax.axis_index("s")
        wid = sid * 2 + half
        lo = wid * WSPAN
        hi = jnp.minimum(lo + WSPAN, N)
        hi_full = jnp.minimum(lo + WSPAN, FULL_END)
        iota = lax.iota(jnp.int32, 16)

        pltpu.sync_copy(tailT_hbm, tail_v)

        def issue(c, sbuf, sem):
            dma_cs = jnp.minimum(lo + c * CHUNK, ALIGN_MAX)
            pltpu.async_copy(tableT_hbm.at[:, pl.ds(dma_cs, CHUNK)], sbuf,
                             sem)

        ring = ((sb0, sem0), (sb1, sem1), (sb2, sem2), (sb3, sem3))
        for b in range(4):
            issue(jnp.int32(b), *ring[b])

        def scan_blk(bk, cnt):
            pltpu.sync_copy(
                idx_hbm.at[pl.ds(pl.multiple_of(bk * 2048, 2048), 2048)],
                cid_v.at[pl.ds(0, 2048)])

            def scan_body(g, cnt):
                ids = cid_v[pl.ds(g * 16, 16)]
                m = (ids >= lo) & (ids < hi)
                plsc.store_compressed(mid_v.at[pl.ds(cnt, 16)], ids, mask=m)
                plsc.store_compressed(
                    mrow_v.at[pl.ds(cnt, 16)],
                    iota + (bk * 2048 + g * 16), mask=m)
                return cnt + jnp.max(plsc.all_reduce_population_count(m))

            return lax.fori_loop(0, 128, scan_body, cnt)

        mcnt = lax.fori_loop(0, B // 2048, scan_blk, jnp.int32(0))

        padded = (mcnt + 127) & ~jnp.int32(127)
        stage16[...] = jnp.full((16,), 0, jnp.int32) + padded
        pltpu.sync_copy(
            stage16,
            counts_hbm.at[pl.ds(pl.multiple_of(wid * 16, 16), 16)])
        region = wid * RCAP

        def flush(fbase):
            rowstart = pl.multiple_of(
                jnp.minimum(region + fbase * 128, TROWS - 128), 128)
            pltpu.sync_copy(ostage.at[pl.ds(0, 128 * 128)],
                            temp_hbm.at[pl.ds(rowstart * 128, 128 * 128)])
            pltpu.sync_copy(oridx.at[pl.ds(0, 128)],
                            rowmap_hbm.at[pl.ds(rowstart, 128)])

        def extract(src_ref, dma_start, cs, ce, st0):
            def win_body(w, st_w):
                bb = w * WIN
                nvec = (jnp.minimum(mcnt - bb, WIN) + 15) >> 4

                def filt(v, ccnt):
                    off = bb + v * 16
                    ids = mid_v[pl.ds(off, 16)]
                    rows = mrow_v[pl.ds(off, 16)]
                    m = ((iota + off) < mcnt) & (ids >= cs) & (ids < ce)
                    plsc.store_compressed(cid_v.at[pl.ds(ccnt, 16)], ids,
                                          mask=m)
                    plsc.store_compressed(crow_v.at[pl.ds(ccnt, 16)], rows,
                                          mask=m)
                    return ccnt + jnp.max(plsc.all_reduce_population_count(m))

                ccnt = lax.fori_loop(0, nvec, filt, jnp.int32(0))

                def grp(g, st):
                    acnt, fbase = st
                    ids = cid_v[pl.ds(g * 16, 16)]
                    rows = crow_v[pl.ds(g * 16, 16)]
                    lm = iota < (ccnt - g * 16)
                    pos = jnp.where(lm, ids - dma_start, 0)
                    prow = jnp.minimum(
                        acnt + plsc.cumsum(lm.astype(jnp.int32)) - 1,
                        OCAP - 1)
                    plsc.store_scatter(oridx, [prow], rows, mask=lm)
                    pbase = prow * 128
                    for d in range(D):
                        dv = jnp.full((16,), d, jnp.int32)
                        vals = plsc.load_gather(src_ref, [dv, pos])
                        plsc.store_scatter(ostage, [pbase + d], vals,
                                           mask=lm)
                    nacnt = acnt + jnp.max(
                        plsc.all_reduce_population_count(lm))
                    do_flush = nacnt >= 128

                    @pl.when(do_flush)
                    def _flush_and_carry():
                        flush(fbase)
                        rem = nacnt - 128

                        def move_row(k, _):
                            src = (128 + k) * 128
                            dst = k * 128
                            for t in range(8):
                                ostage[pl.ds(dst + t * 16, 16)] = (
                                    ostage[pl.ds(src + t * 16, 16)])
                            return _

                        lax.fori_loop(0, rem, move_row, 0)
                        mv = oridx[pl.ds(128, 16)]
                        oridx[pl.ds(0, 16)] = mv

                    acnt = jnp.where(do_flush, nacnt - 128, nacnt)
                    fbase = jnp.where(do_flush, fbase + 1, fbase)
                    return (acnt, fbase)

                return lax.fori_loop(0, (ccnt + 15) >> 4, grp, st_w)

            return lax.fori_loop(0, (mcnt + (WIN - 1)) >> 11, win_body, st0)

        def quad(i, st):
            for bb, (sbuf, sem) in enumerate(ring):
                c = i * 4 + bb
                dma_cs = jnp.minimum(lo + c * CHUNK, ALIGN_MAX)
                pltpu.make_async_copy(
                    tableT_hbm.at[:, pl.ds(dma_cs, CHUNK)], sbuf, sem).wait()
                cs = lo + c * CHUNK
                ce = jnp.minimum(cs + CHUNK, hi_full)
                st = extract(sbuf, dma_cs, cs, ce, st)

                @pl.when(c + 4 < NCHUNKS)
                def _issue_next(sbuf=sbuf, sem=sem, c=c):
                    issue(c + 4, sbuf, sem)

            return st

        st = lax.fori_loop(0, NCHUNKS // 4, quad,
                           (jnp.int32(0), jnp.int32(0)))

        acnt, fbase = extract(tail_v, jnp.int32(FULL_END),
                              jnp.int32(FULL_END), jnp.int32(N), st)

        @pl.when(acnt > 0)
        def _final_flush():
            for s in range(8):
                l = iota + s * 16
                cur = oridx[pl.ds(s * 16, 16)]
                oridx[pl.ds(s * 16, 16)] = jnp.where(l < acnt, cur, DUMP)
            flush(fbase)

    @functools.partial(
        pl.kernel,
        mesh=mesh,
        out_type=jax.ShapeDtypeStruct((B, 128), jnp.float32),
        scratch_types=[
            pltpu.VMEM((2048,), jnp.int32),
            pltpu.VMEM((2048,), jnp.int32),
            pltpu.VMEM((512 + 16,), jnp.int32),
            pltpu.VMEM((128, 128), jnp.float32),
            pltpu.VMEM((NW * 16,), jnp.int32),
            pltpu.SemaphoreType.DMA,
            pltpu.SemaphoreType.DMA,
            pltpu.SemaphoreType.DMA,
        ],
        compiler_params=pltpu.CompilerParams(
            use_tc_tiling_on_sc=False, needs_layout_passes=False),
    )
    def k2(temp_hbm, rowmap_hbm, counts_hbm, out_hbm, rbufa, rbufb, inv,
           gbuf, cnt_v, sem, rsem_a, rsem_b):
        wid = lax.axis_index("s") * 2 + lax.axis_index("c")
        iota = lax.iota(jnp.int32, 16)
        rlo = wid * 512
        pltpu.sync_copy(counts_hbm, cnt_v)
        for i in range(33):
            inv[pl.ds(i * 16, 16)] = jnp.zeros((16,), jnp.int32)

        def rm_slice(c, bb):
            return rowmap_hbm.at[pl.ds(
                pl.multiple_of(c * RCAP + bb * 2048, 2048), 2048)]

        pltpu.async_copy(rm_slice(0, 0), rbufa, rsem_a)
        for c in range(NW):
            rbuf, rsem = (rbufa, rsem_a) if c % 2 == 0 else (rbufb, rsem_b)
            nxt, nsem = (rbufb, rsem_b) if c % 2 == 0 else (rbufa, rsem_a)
            nent = jnp.clip(jnp.max(cnt_v[pl.ds(c * 16, 16)]), 0, RCAP)
            pltpu.make_async_copy(rm_slice(c, 0), rbuf, rsem).wait()
            if c + 1 < NW:
                pltpu.async_copy(rm_slice(c + 1, 0), nxt, nsem)

            def blk(bb, _, c=c, nent=nent, rbuf=rbuf):
                @pl.when(bb > 0)
                def _fetch(bb=bb, c=c, rbuf=rbuf):
                    pltpu.sync_copy(rm_slice(c, bb), rbuf)

                nv = (jnp.minimum(nent - bb * 2048, 2048) + 15) >> 4

                def vec(v, __, bb=bb, c=c, nent=nent, rbuf=rbuf):
                    l = bb * 2048 + v * 16
                    rm = rbuf[pl.ds(v * 16, 16)]
                    m = ((l + iota) < nent) & (rm >= rlo) & (rm < rlo + 512)
                    eidx = c * RCAP + l + iota
                    plsc.store_scatter(
                        inv, [jnp.where(m, rm - rlo, 512)], eidx, mask=m)
                    return __

                lax.fori_loop(0, nv, vec, 0)
                return _

            lax.fori_loop(0, (nent + 2047) >> 11, blk, 0)

        for sub in range(4):
            pltpu.async_copy(
                temp_hbm.at[inv.at[pl.ds(sub * 128, 128)]], gbuf, sem).wait()
            pltpu.sync_copy(gbuf, out_hbm.at[pl.ds(rlo + sub * 128, 128)])

    tailT = lax.slice(table, (FULL_END, 0), (N, D)).T
    temp, rowmap, counts = k1(node_ids.astype(jnp.int32), table.T, tailT)
    wide = k2(temp.reshape(TROWS, 128), rowmap, counts)
    return wide[:, :D]


# speedup vs baseline: 7.2848x; 1.0071x over previous
import functools

import jax
import jax.numpy as jnp
from jax import lax
from jax.experimental import pallas as pl
from jax.experimental.pallas import tpu as pltpu
from jax.experimental.pallas import tpu_sc as plsc

N = 1000000
D = 64
B = 16384
NW = 32
SPW = 245
WSPAN = SPW * 128
FULL_END = (N // 128) * 128
TAIL = N - FULL_END
CHUNK = 256
NCHUNKS = 124
ALIGN_MAX = FULL_END - CHUNK
WIN = 2048
RCAP = B + 16 * 128
TROWS = NW * RCAP
DUMP = B
OCAP = 144


def kernel(node_ids, table):
    mesh = plsc.VectorSubcoreMesh(core_axis_name="c", subcore_axis_name="s")

    @functools.partial(
        pl.kernel,
        mesh=mesh,
        out_type=[
            jax.ShapeDtypeStruct((TROWS * 128,), jnp.float32),
            jax.ShapeDtypeStruct((TROWS,), jnp.int32),
            jax.ShapeDtypeStruct((NW * 16,), jnp.int32),
        ],
        scratch_types=[
            pltpu.VMEM((B + 16,), jnp.int32),
            pltpu.VMEM((B + 16,), jnp.int32),
            pltpu.VMEM((WIN + 16,), jnp.int32),
            pltpu.VMEM((WIN + 16,), jnp.int32),
            pltpu.VMEM((D, CHUNK), jnp.float32),
            pltpu.VMEM((D, CHUNK), jnp.float32),
            pltpu.VMEM((D, CHUNK), jnp.float32),
            pltpu.VMEM((D, CHUNK), jnp.float32),
            pltpu.VMEM((D, TAIL), jnp.float32),
            pltpu.VMEM((OCAP * 128,), jnp.float32),
            pltpu.VMEM((OCAP,), jnp.int32),
            pltpu.VMEM((16,), jnp.int32),
            pltpu.SemaphoreType.DMA,
            pltpu.SemaphoreType.DMA,
            pltpu.SemaphoreType.DMA,
            pltpu.SemaphoreType.DMA,
        ],
        compiler_params=pltpu.CompilerParams(needs_layout_passes=False),
    )
    def k1(idx_hbm, tableT_hbm, tailT_hbm, temp_hbm, rowmap_hbm, counts_hbm,
           mid_v, mrow_v, cid_v, crow_v, sb0, sb1, sb2, sb3, tail_v,
           ostage, oridx, stage16, sem0, sem1, sem2, sem3):
        half = lax.axis_index("c")
        sid = lax.axis_index("s")
        wid = sid * 2 + half
        lo = wid * WSPAN
        hi = jnp.minimum(lo + WSPAN, N)
        hi_full = jnp.minimum(lo + WSPAN, FULL_END)
        iota = lax.iota(jnp.int32, 16)

        pltpu.sync_copy(tailT_hbm, tail_v)

        def issue(c, sbuf, sem):
            dma_cs = jnp.minimum(lo + c * CHUNK, ALIGN_MAX)
            pltpu.async_copy(tableT_hbm.at[:, pl.ds(dma_cs, CHUNK)], sbuf,
                             sem)

        ring = ((sb0, sem0), (sb1, sem1), (sb2, sem2), (sb3, sem3))
        for b in range(4):
            issue(jnp.int32(b), *ring[b])

        def scan_blk(bk, cnt):
            pltpu.sync_copy(
                idx_hbm.at[pl.ds(pl.multiple_of(bk * 2048, 2048), 2048)],
                cid_v.at[pl.ds(0, 2048)])

            def scan_body(g, cnt):
                ids = cid_v[pl.ds(g * 16, 16)]
                m = (ids >= lo) & (ids < hi)
                plsc.store_compressed(mid_v.at[pl.ds(cnt, 16)], ids, mask=m)
                plsc.store_compressed(
                    mrow_v.at[pl.ds(cnt, 16)],
                    iota + (bk * 2048 + g * 16), mask=m)
                return cnt + jnp.max(plsc.all_reduce_population_count(m))

            return lax.fori_loop(0, 128, scan_body, cnt)

        mcnt = lax.fori_loop(0, B // 2048, scan_blk, jnp.int32(0))

        padded = (mcnt + 127) & ~jnp.int32(127)
        stage16[...] = jnp.full((16,), 0, jnp.int32) + padded
        pltpu.sync_copy(
            stage16,
            counts_hbm.at[pl.ds(pl.multiple_of(wid * 16, 16), 16)])
        region = wid * RCAP

        def flush(fbase):
            rowstart = pl.multiple_of(
                jnp.minimum(region + fbase * 128, TROWS - 128), 128)
            pltpu.sync_copy(ostage.at[pl.ds(0, 128 * 128)],
                            temp_hbm.at[pl.ds(rowstart * 128, 128 * 128)])
            pltpu.sync_copy(oridx.at[pl.ds(0, 128)],
                            rowmap_hbm.at[pl.ds(rowstart, 128)])

        def extract(src_ref, dma_start, cs, ce, st0):
            def win_body(w, st_w):
                bb = w * WIN
                nvec = (jnp.minimum(mcnt - bb, WIN) + 15) >> 4

                def filt(v, ccnt):
                    off = bb + v * 16
                    ids = mid_v[pl.ds(off, 16)]
                    rows = mrow_v[pl.ds(off, 16)]
                    m = ((iota + off) < mcnt) & (ids >= cs) & (ids < ce)
                    plsc.store_compressed(cid_v.at[pl.ds(ccnt, 16)], ids,
                                          mask=m)
                    plsc.store_compressed(crow_v.at[pl.ds(ccnt, 16)], rows,
                                          mask=m)
                    return ccnt + jnp.max(plsc.all_reduce_population_count(m))

                ccnt = lax.fori_loop(0, nvec, filt, jnp.int32(0))

                def grp(g, st):
                    acnt, fbase = st
                    ids = cid_v[pl.ds(g * 16, 16)]
                    rows = crow_v[pl.ds(g * 16, 16)]
                    lm = iota < (ccnt - g * 16)
                    pos = jnp.where(lm, ids - dma_start, 0)
                    prow = jnp.minimum(
                        acnt + plsc.cumsum(lm.astype(jnp.int32)) - 1,
                        OCAP - 1)
                    plsc.store_scatter(oridx, [prow], rows, mask=lm)
                    pbase = prow * 128
                    for d in range(D):
                        dv = jnp.full((16,), d, jnp.int32)
                        vals = plsc.load_gather(src_ref, [dv, pos])
                        plsc.store_scatter(ostage, [pbase + d], vals,
                                           mask=lm)
                    nacnt = acnt + jnp.max(
                        plsc.all_reduce_population_count(lm))
                    do_flush = nacnt >= 128

                    @pl.when(do_flush)
                    def _flush_and_carry():
                        flush(fbase)
                        rem = nacnt - 128

                        def move_row(k, _):
                            src = (128 + k) * 128
                            dst = k * 128
                            for t in range(8):
                                ostage[pl.ds(dst + t * 16, 16)] = (
                                    ostage[pl.ds(src + t * 16, 16)])
                            return _

                        lax.fori_loop(0, rem, move_row, 0)
                        mv = oridx[pl.ds(128, 16)]
                        oridx[pl.ds(0, 16)] = mv

                    acnt = jnp.where(do_flush, nacnt - 128, nacnt)
                    fbase = jnp.where(do_flush, fbase + 1, fbase)
                    return (acnt, fbase)

                return lax.fori_loop(0, (ccnt + 15) >> 4, grp, st_w)

            return lax.fori_loop(0, (mcnt + (WIN - 1)) >> 11, win_body, st0)

        def quad(i, st):
            for bb, (sbuf, sem) in enumerate(ring):
                c = i * 4 + bb
                dma_cs = jnp.minimum(lo + c * CHUNK, ALIGN_MAX)
                pltpu.make_async_copy(
                    tableT_hbm.at[:, pl.ds(dma_cs, CHUNK)], sbuf, sem).wait()
                cs = lo + c * CHUNK
                ce = jnp.minimum(cs + CHUNK, hi_full)
                st = extract(sbuf, dma_cs, cs, ce, st)

                @pl.when(c + 4 < NCHUNKS)
                def _issue_next(sbuf=sbuf, sem=sem, c=c):
                    issue(c + 4, sbuf, sem)

            return st

        st = lax.fori_loop(0, NCHUNKS // 4, quad,
                           (jnp.int32(0), jnp.int32(0)))

        acnt, fbase = extract(tail_v, jnp.int32(FULL_END),
                              jnp.int32(FULL_END), jnp.int32(N), st)

        @pl.when(acnt > 0)
        def _final_flush():
            for s in range(8):
                l = iota + s * 16
                cur = oridx[pl.ds(s * 16, 16)]
                oridx[pl.ds(s * 16, 16)] = jnp.where(l < acnt, cur, DUMP)
            flush(fbase)

    @functools.partial(
        pl.kernel,
        mesh=mesh,
        out_type=jax.ShapeDtypeStruct((B, 128), jnp.float32),
        scratch_types=[
            pltpu.VMEM((2048,), jnp.int32),
            pltpu.VMEM((2048,), jnp.int32),
            pltpu.VMEM((512 + 16,), jnp.int32),
            pltpu.VMEM((128, 128), jnp.float32),
            pltpu.VMEM((128, 128), jnp.float32),
            pltpu.VMEM((NW * 16,), jnp.int32),
            pltpu.SemaphoreType.DMA,
            pltpu.SemaphoreType.DMA,
            pltpu.SemaphoreType.DMA,
            pltpu.SemaphoreType.DMA,
        ],
        compiler_params=pltpu.CompilerParams(
            use_tc_tiling_on_sc=False, needs_layout_passes=False),
    )
    def k2(temp_hbm, rowmap_hbm, counts_hbm, out_hbm, rbufa, rbufb, inv,
           gbufa, gbufb, cnt_v, sem, sem_b, rsem_a, rsem_b):
        wid = lax.axis_index("s") * 2 + lax.axis_index("c")
        iota = lax.iota(jnp.int32, 16)
        rlo = wid * 512
        pltpu.sync_copy(counts_hbm, cnt_v)
        for i in range(33):
            inv[pl.ds(i * 16, 16)] = jnp.zeros((16,), jnp.int32)

        def rm_slice(c, bb):
            return rowmap_hbm.at[pl.ds(
                pl.multiple_of(c * RCAP + bb * 2048, 2048), 2048)]

        pltpu.async_copy(rm_slice(0, 0), rbufa, rsem_a)
        for c in range(NW):
            rbuf, rsem = (rbufa, rsem_a) if c % 2 == 0 else (rbufb, rsem_b)
            nxt, nsem = (rbufb, rsem_b) if c % 2 == 0 else (rbufa, rsem_a)
            nent = jnp.clip(jnp.max(cnt_v[pl.ds(c * 16, 16)]), 0, RCAP)
            pltpu.make_async_copy(rm_slice(c, 0), rbuf, rsem).wait()
            if c + 1 < NW:
                pltpu.async_copy(rm_slice(c + 1, 0), nxt, nsem)

            def blk(bb, _, c=c, nent=nent, rbuf=rbuf):
                @pl.when(bb > 0)
                def _fetch(bb=bb, c=c, rbuf=rbuf):
                    pltpu.sync_copy(rm_slice(c, bb), rbuf)

                nv = (jnp.minimum(nent - bb * 2048, 2048) + 15) >> 4

                def vec(v, __, bb=bb, c=c, nent=nent, rbuf=rbuf):
                    l = bb * 2048 + v * 16
                    rm = rbuf[pl.ds(v * 16, 16)]
                    m = ((l + iota) < nent) & (rm >= rlo) & (rm < rlo + 512)
                    eidx = c * RCAP + l + iota
                    plsc.store_scatter(
                        inv, [jnp.where(m, rm - rlo, 512)], eidx, mask=m)
                    return __

                lax.fori_loop(0, nv, vec, 0)
                return _

            lax.fori_loop(0, (nent + 2047) >> 11, blk, 0)

        grings = ((gbufa, sem), (gbufb, sem_b))
        pltpu.async_copy(temp_hbm.at[inv.at[pl.ds(0, 128)]], gbufa, sem)
        for sub in range(4):
            gbuf, gsem = grings[sub % 2]
            pltpu.make_async_copy(
                temp_hbm.at[inv.at[pl.ds(sub * 128, 128)]], gbuf, gsem).wait()
            if sub + 1 < 4:
                nbuf, nsem = grings[(sub + 1) % 2]
                pltpu.async_copy(
                    temp_hbm.at[inv.at[pl.ds((sub + 1) * 128, 128)]], nbuf,
                    nsem)
            pltpu.sync_copy(gbuf, out_hbm.at[pl.ds(rlo + sub * 128, 128)])

    tailT = lax.slice(table, (FULL_END, 0), (N, D)).T
    temp, rowmap, counts = k1(node_ids.astype(jnp.int32), table.T, tailT)
    wide = k2(temp.reshape(TROWS, 128), rowmap, counts)
    return wide[:, :D]
